# Initial kernel scaffold; baseline (speedup 1.0000x reference)
import functools

import jax
import jax.numpy as jnp
from jax import lax
from jax.experimental import pallas as pl
from jax.experimental.pallas import tpu as pltpu
from jax.experimental.pallas import tpu_sc as plsc

N_NODES = 10000
NPAD = 10240
D_FEAT = 128
N_CLASSES = 64
N_EDGES = 320000
NC = 2
NS = 16
NW = NC * NS
CHUNK = 128
NCH = 80
EPAD = NW * NCH * CHUNK
TRASH = N_NODES + 200
RPT = NPAD // NS

_mesh = plsc.VectorSubcoreMesh(core_axis_name="c", subcore_axis_name="s")


@functools.partial(
    pl.kernel,
    out_type=jax.ShapeDtypeStruct((NC, NPAD, 16), jnp.float32),
    mesh=_mesh,
    scratch_types=[
        pltpu.VMEM((NCH, CHUNK), jnp.int32),
        pltpu.VMEM((CHUNK, 16), jnp.float32),
        pltpu.VMEM_SHARED((NPAD, 16), jnp.float32),
    ],
)
def _deg_kernel(dst_hbm, ones_hbm, zeros_hbm, out_hbm, dst_v, ones_v, acc):
    c = lax.axis_index("c")
    s = lax.axis_index("s")
    w = c * NS + s
    pltpu.sync_copy(dst_hbm.at[w], dst_v)
    pltpu.sync_copy(ones_hbm, ones_v)
    pltpu.sync_copy(zeros_hbm.at[pl.ds(s * RPT, RPT)], acc.at[pl.ds(s * RPT, RPT)])
    plsc.subcore_barrier()

    def body(j, carry):
        pltpu.sync_copy(ones_v, acc.at[dst_v.at[j]], add=True)
        return carry

    lax.fori_loop(0, NCH, body, 0)
    plsc.subcore_barrier()
    pltpu.sync_copy(acc.at[pl.ds(s * RPT, RPT)], out_hbm.at[c, pl.ds(s * RPT, RPT)])


@functools.partial(
    pl.kernel,
    out_type=jax.ShapeDtypeStruct((NC, NPAD, N_CLASSES), jnp.float32),
    mesh=_mesh,
    scratch_types=[
        pltpu.VMEM((NCH, CHUNK), jnp.int32),
        pltpu.VMEM((NCH, CHUNK), jnp.int32),
        pltpu.VMEM((CHUNK, N_CLASSES), jnp.float32),
        pltpu.VMEM((CHUNK, N_CLASSES), jnp.float32),
        pltpu.SemaphoreType.DMA,
        pltpu.SemaphoreType.DMA,
        pltpu.VMEM_SHARED((NPAD, N_CLASSES), jnp.float32),
    ],
)
def _hop_kernel(g_hbm, src_hbm, dst_hbm, zeros_hbm, out_hbm,
                src_v, dst_v, buf_a, buf_b, sem_a, sem_b, acc):
    c = lax.axis_index("c")
    s = lax.axis_index("s")
    w = c * NS + s
    pltpu.sync_copy(src_hbm.at[w], src_v)
    pltpu.sync_copy(dst_hbm.at[w], dst_v)
    pltpu.sync_copy(zeros_hbm.at[pl.ds(s * RPT, RPT)], acc.at[pl.ds(s * RPT, RPT)])
    plsc.subcore_barrier()

    pltpu.async_copy(g_hbm.at[src_v.at[0]], buf_a, sem_a)

    def body(p, carry):
        j = 2 * p
        pltpu.async_copy(g_hbm.at[src_v.at[j + 1]], buf_b, sem_b)
        pltpu.make_async_copy(g_hbm.at[src_v.at[j]], buf_a, sem_a).wait()
        pltpu.sync_copy(buf_a, acc.at[dst_v.at[j]], add=True)

        @pl.when(j + 2 < NCH)
        def _prefetch():
            pltpu.async_copy(g_hbm.at[src_v.at[j + 2]], buf_a, sem_a)

        pltpu.make_async_copy(g_hbm.at[src_v.at[j + 1]], buf_b, sem_b).wait()
        pltpu.sync_copy(buf_b, acc.at[dst_v.at[j + 1]], add=True)
        return carry

    lax.fori_loop(0, NCH // 2, body, 0)
    plsc.subcore_barrier()
    pltpu.sync_copy(acc.at[pl.ds(s * RPT, RPT)], out_hbm.at[c, pl.ds(s * RPT, RPT)])


_RB = 640


def _scale_body(degp_ref, x_ref, w_ref, g0_ref, dinv_ref):
    deg = degp_ref[0, :, 0:1] + degp_ref[1, :, 0:1] + 1.0
    dinv = lax.rsqrt(deg)
    h0 = jnp.dot(x_ref[...], w_ref[...], preferred_element_type=jnp.float32)
    g0_ref[...] = h0 * dinv
    dinv_ref[...] = jnp.broadcast_to(dinv, (_RB, N_CLASSES))


_scale_call = pl.pallas_call(
    _scale_body,
    grid=(NPAD // _RB,),
    in_specs=[
        pl.BlockSpec((NC, _RB, 16), lambda i: (0, i, 0)),
        pl.BlockSpec((_RB, D_FEAT), lambda i: (i, 0)),
        pl.BlockSpec((D_FEAT, N_CLASSES), lambda i: (0, 0)),
    ],
    out_specs=[
        pl.BlockSpec((_RB, N_CLASSES), lambda i: (i, 0)),
        pl.BlockSpec((_RB, N_CLASSES), lambda i: (i, 0)),
    ],
    out_shape=[
        jax.ShapeDtypeStruct((NPAD, N_CLASSES), jnp.float32),
        jax.ShapeDtypeStruct((NPAD, N_CLASSES), jnp.float32),
    ],
)


def _combine_body(p_ref, g0_ref, dinv_ref, g1_ref):
    t = p_ref[0] + p_ref[1] + g0_ref[...]
    dv = dinv_ref[...]
    g1_ref[...] = dv * dv * t


_combine_call = pl.pallas_call(
    _combine_body,
    grid=(NPAD // _RB,),
    in_specs=[
        pl.BlockSpec((NC, _RB, N_CLASSES), lambda i: (0, i, 0)),
        pl.BlockSpec((_RB, N_CLASSES), lambda i: (i, 0)),
        pl.BlockSpec((_RB, N_CLASSES), lambda i: (i, 0)),
    ],
    out_specs=pl.BlockSpec((_RB, N_CLASSES), lambda i: (i, 0)),
    out_shape=jax.ShapeDtypeStruct((NPAD, N_CLASSES), jnp.float32),
)

_FB = 400


def _final_body(q_ref, g1_ref, dinv_ref, b_ref, out_ref):
    logits = dinv_ref[...] * (q_ref[0] + q_ref[1] + g1_ref[...]) + b_ref[...]
    m = jnp.max(logits, axis=1, keepdims=True)
    ex = jnp.exp(logits - m)
    out_ref[...] = logits - m - jnp.log(jnp.sum(ex, axis=1, keepdims=True))


_final_call = pl.pallas_call(
    _final_body,
    grid=(N_NODES // _FB,),
    in_specs=[
        pl.BlockSpec((NC, _FB, N_CLASSES), lambda i: (0, i, 0)),
        pl.BlockSpec((_FB, N_CLASSES), lambda i: (i, 0)),
        pl.BlockSpec((_FB, N_CLASSES), lambda i: (i, 0)),
        pl.BlockSpec((1, N_CLASSES), lambda i: (0, 0)),
    ],
    out_specs=pl.BlockSpec((_FB, N_CLASSES), lambda i: (i, 0)),
    out_shape=jax.ShapeDtypeStruct((N_NODES, N_CLASSES), jnp.float32),
)


def kernel(x, edge_index, W, b):
    src = edge_index[0].astype(jnp.int32)
    dst = edge_index[1].astype(jnp.int32)
    pad_e = EPAD - N_EDGES
    src_p = jnp.concatenate(
        [src, jnp.zeros((pad_e,), jnp.int32)]).reshape(NW, NCH, CHUNK)
    dst_p = jnp.concatenate(
        [dst, jnp.full((pad_e,), TRASH, jnp.int32)]).reshape(NW, NCH, CHUNK)
    x_p = jnp.concatenate(
        [x, jnp.zeros((NPAD - N_NODES, D_FEAT), jnp.float32)])
    ones16 = jnp.ones((CHUNK, 16), jnp.float32)
    zeros16 = jnp.zeros((NPAD, 16), jnp.float32)
    zeros64 = jnp.zeros((NPAD, N_CLASSES), jnp.float32)

    degp = _deg_kernel(dst_p, ones16, zeros16)
    g0, dinv = _scale_call(degp, x_p, W)
    p = _hop_kernel(g0, src_p, dst_p, zeros64)
    g1 = _combine_call(p, g0, dinv)
    q = _hop_kernel(g1, src_p, dst_p, zeros64)
    return _final_call(q, g1, dinv, b.reshape(1, N_CLASSES))


# speedup vs baseline: 14.8796x; 14.8796x over previous
import functools

import jax
import jax.numpy as jnp
from jax import lax
from jax.experimental import pallas as pl
from jax.experimental.pallas import tpu as pltpu
from jax.experimental.pallas import tpu_sc as plsc

N_NODES = 10000
NPAD = 10240
D_FEAT = 128
N_CLASSES = 64
N_EDGES = 320000
NC = 2
NS = 16
NW = NC * NS
CHUNK = 128
NCH = 80
EPAD = NW * NCH * CHUNK
TRASH = N_NODES + 200
RPT = NPAD // NS

_mesh = plsc.VectorSubcoreMesh(core_axis_name="c", subcore_axis_name="s")


@functools.partial(
    pl.kernel,
    out_type=jax.ShapeDtypeStruct((NC, NPAD, 16), jnp.float32),
    mesh=_mesh,
    scratch_types=[
        pltpu.VMEM((NCH, CHUNK), jnp.int32),
        pltpu.VMEM((CHUNK, 16), jnp.float32),
        pltpu.VMEM_SHARED((NPAD, 16), jnp.float32),
    ],
    compiler_params=pltpu.CompilerParams(use_tc_tiling_on_sc=False),
)
def _deg_kernel(dst_hbm, ones_hbm, zeros_hbm, out_hbm, dst_v, ones_v, acc):
    c = lax.axis_index("c")
    s = lax.axis_index("s")
    w = c * NS + s
    pltpu.sync_copy(dst_hbm.at[w], dst_v)
    pltpu.sync_copy(ones_hbm, ones_v)
    pltpu.sync_copy(zeros_hbm.at[pl.ds(s * RPT, RPT)], acc.at[pl.ds(s * RPT, RPT)])
    plsc.subcore_barrier()

    def body(j, carry):
        pltpu.sync_copy(ones_v, acc.at[dst_v.at[j]], add=True)
        return carry

    lax.fori_loop(0, NCH, body, 0)
    plsc.subcore_barrier()
    pltpu.sync_copy(acc.at[pl.ds(s * RPT, RPT)], out_hbm.at[c, pl.ds(s * RPT, RPT)])


@functools.partial(
    pl.kernel,
    out_type=jax.ShapeDtypeStruct((NC, NPAD, N_CLASSES), jnp.float32),
    mesh=_mesh,
    scratch_types=[
        pltpu.VMEM((NCH, CHUNK), jnp.int32),
        pltpu.VMEM((NCH, CHUNK), jnp.int32),
        pltpu.VMEM((CHUNK, N_CLASSES), jnp.float32),
        pltpu.VMEM((CHUNK, N_CLASSES), jnp.float32),
        pltpu.SemaphoreType.DMA,
        pltpu.SemaphoreType.DMA,
        pltpu.VMEM_SHARED((NPAD, N_CLASSES), jnp.float32),
    ],
    compiler_params=pltpu.CompilerParams(use_tc_tiling_on_sc=False),
)
def _hop_kernel(g_hbm, src_hbm, dst_hbm, zeros_hbm, out_hbm,
                src_v, dst_v, buf_a, buf_b, sem_a, sem_b, acc):
    c = lax.axis_index("c")
    s = lax.axis_index("s")
    w = c * NS + s
    pltpu.sync_copy(src_hbm.at[w], src_v)
    pltpu.sync_copy(dst_hbm.at[w], dst_v)
    pltpu.sync_copy(zeros_hbm.at[pl.ds(s * RPT, RPT)], acc.at[pl.ds(s * RPT, RPT)])
    plsc.subcore_barrier()

    pltpu.async_copy(g_hbm.at[src_v.at[0]], buf_a, sem_a)

    def body(p, carry):
        j = 2 * p
        pltpu.async_copy(g_hbm.at[src_v.at[j + 1]], buf_b, sem_b)
        pltpu.make_async_copy(g_hbm.at[src_v.at[j]], buf_a, sem_a).wait()
        pltpu.sync_copy(buf_a, acc.at[dst_v.at[j]], add=True)

        @pl.when(j + 2 < NCH)
        def _prefetch():
            pltpu.async_copy(g_hbm.at[src_v.at[j + 2]], buf_a, sem_a)

        pltpu.make_async_copy(g_hbm.at[src_v.at[j + 1]], buf_b, sem_b).wait()
        pltpu.sync_copy(buf_b, acc.at[dst_v.at[j + 1]], add=True)
        return carry

    lax.fori_loop(0, NCH // 2, body, 0)
    plsc.subcore_barrier()
    pltpu.sync_copy(acc.at[pl.ds(s * RPT, RPT)], out_hbm.at[c, pl.ds(s * RPT, RPT)])


_RB = 640


def _scale_body(degp_ref, x_ref, w_ref, g0_ref, dinv_ref):
    deg = degp_ref[0, :, 0:1] + degp_ref[1, :, 0:1] + 1.0
    dinv = lax.rsqrt(deg)
    h0 = jnp.dot(x_ref[...], w_ref[...], preferred_element_type=jnp.float32)
    g0_ref[...] = h0 * dinv
    dinv_ref[...] = jnp.broadcast_to(dinv, (_RB, N_CLASSES))


_scale_call = pl.pallas_call(
    _scale_body,
    grid=(NPAD // _RB,),
    in_specs=[
        pl.BlockSpec((NC, _RB, 16), lambda i: (0, i, 0)),
        pl.BlockSpec((_RB, D_FEAT), lambda i: (i, 0)),
        pl.BlockSpec((D_FEAT, N_CLASSES), lambda i: (0, 0)),
    ],
    out_specs=[
        pl.BlockSpec((_RB, N_CLASSES), lambda i: (i, 0)),
        pl.BlockSpec((_RB, N_CLASSES), lambda i: (i, 0)),
    ],
    out_shape=[
        jax.ShapeDtypeStruct((NPAD, N_CLASSES), jnp.float32),
        jax.ShapeDtypeStruct((NPAD, N_CLASSES), jnp.float32),
    ],
)


def _combine_body(p_ref, g0_ref, dinv_ref, g1_ref):
    t = p_ref[0] + p_ref[1] + g0_ref[...]
    dv = dinv_ref[...]
    g1_ref[...] = dv * dv * t


_combine_call = pl.pallas_call(
    _combine_body,
    grid=(NPAD // _RB,),
    in_specs=[
        pl.BlockSpec((NC, _RB, N_CLASSES), lambda i: (0, i, 0)),
        pl.BlockSpec((_RB, N_CLASSES), lambda i: (i, 0)),
        pl.BlockSpec((_RB, N_CLASSES), lambda i: (i, 0)),
    ],
    out_specs=pl.BlockSpec((_RB, N_CLASSES), lambda i: (i, 0)),
    out_shape=jax.ShapeDtypeStruct((NPAD, N_CLASSES), jnp.float32),
)

_FB = 400


def _final_body(q_ref, g1_ref, dinv_ref, b_ref, out_ref):
    logits = dinv_ref[...] * (q_ref[0] + q_ref[1] + g1_ref[...]) + b_ref[...]
    m = jnp.max(logits, axis=1, keepdims=True)
    ex = jnp.exp(logits - m)
    out_ref[...] = logits - m - jnp.log(jnp.sum(ex, axis=1, keepdims=True))


_final_call = pl.pallas_call(
    _final_body,
    grid=(N_NODES // _FB,),
    in_specs=[
        pl.BlockSpec((NC, _FB, N_CLASSES), lambda i: (0, i, 0)),
        pl.BlockSpec((_FB, N_CLASSES), lambda i: (i, 0)),
        pl.BlockSpec((_FB, N_CLASSES), lambda i: (i, 0)),
        pl.BlockSpec((1, N_CLASSES), lambda i: (0, 0)),
    ],
    out_specs=pl.BlockSpec((_FB, N_CLASSES), lambda i: (i, 0)),
    out_shape=jax.ShapeDtypeStruct((N_NODES, N_CLASSES), jnp.float32),
)


def kernel(x, edge_index, W, b):
    src = edge_index[0].astype(jnp.int32)
    dst = edge_index[1].astype(jnp.int32)
    pad_e = EPAD - N_EDGES
    src_p = jnp.concatenate(
        [src, jnp.zeros((pad_e,), jnp.int32)]).reshape(NW, NCH, CHUNK)
    dst_p = jnp.concatenate(
        [dst, jnp.full((pad_e,), TRASH, jnp.int32)]).reshape(NW, NCH, CHUNK)
    x_p = jnp.concatenate(
        [x, jnp.zeros((NPAD - N_NODES, D_FEAT), jnp.float32)])
    ones16 = jnp.ones((CHUNK, 16), jnp.float32)
    zeros16 = jnp.zeros((NPAD, 16), jnp.float32)
    zeros64 = jnp.zeros((NPAD, N_CLASSES), jnp.float32)

    degp = _deg_kernel(dst_p, ones16, zeros16)
    g0, dinv = _scale_call(degp, x_p, W)
    p = _hop_kernel(g0, src_p, dst_p, zeros64)
    g1 = _combine_call(p, g0, dinv)
    q = _hop_kernel(g1, src_p, dst_p, zeros64)
    return _final_call(q, g1, dinv, b.reshape(1, N_CLASSES))


# speedup vs baseline: 25.3574x; 1.7042x over previous
import functools

import jax
import jax.numpy as jnp
from jax import lax
from jax.experimental import pallas as pl
from jax.experimental.pallas import tpu as pltpu
from jax.experimental.pallas import tpu_sc as plsc

N_NODES = 10000
NPAD = 10240
D_FEAT = 128
N_CLASSES = 64
N_EDGES = 320000
NC = 2
NS = 16
CHUNK = 128
NCH0 = 120
NCH1 = 38
TCH = NS * (NCH0 + NCH1)
EPAD = TCH * CHUNK
TRASH = N_NODES + 200
RPT = NPAD // NS

_mesh = plsc.VectorSubcoreMesh(core_axis_name="c", subcore_axis_name="s")


def _chunk_base(c, s):
    return jnp.where(c == 0, s * NCH0, NS * NCH0 + s * NCH1)


@functools.partial(
    pl.kernel,
    out_type=jax.ShapeDtypeStruct((NC, NPAD, 16), jnp.float32),
    mesh=_mesh,
    scratch_types=[
        pltpu.VMEM((NCH0, CHUNK), jnp.int32),
        pltpu.VMEM((CHUNK, 16), jnp.float32),
        pltpu.VMEM_SHARED((NPAD, 16), jnp.float32),
    ],
    compiler_params=pltpu.CompilerParams(use_tc_tiling_on_sc=False),
)
def _deg_kernel(dst_hbm, ones_hbm, zeros_hbm, out_hbm, dst_v, ones_v, acc):
    c = lax.axis_index("c")
    s = lax.axis_index("s")
    base = _chunk_base(c, s)
    pltpu.sync_copy(ones_hbm, ones_v)
    pltpu.sync_copy(zeros_hbm.at[pl.ds(s * RPT, RPT)], acc.at[pl.ds(s * RPT, RPT)])

    @pl.when(c == 0)
    def _load0():
        pltpu.sync_copy(dst_hbm.at[pl.ds(base, NCH0)], dst_v)

    @pl.when(c == 1)
    def _load1():
        pltpu.sync_copy(dst_hbm.at[pl.ds(base, NCH1)], dst_v.at[pl.ds(0, NCH1)])

    plsc.subcore_barrier()

    def body(j, carry):
        pltpu.sync_copy(ones_v, acc.at[dst_v.at[j]], add=True)
        return carry

    @pl.when(c == 0)
    def _scat0():
        lax.fori_loop(0, NCH0, body, 0)

    @pl.when(c == 1)
    def _scat1():
        lax.fori_loop(0, NCH1, body, 0)

    plsc.subcore_barrier()
    pltpu.sync_copy(acc.at[pl.ds(s * RPT, RPT)], out_hbm.at[c, pl.ds(s * RPT, RPT)])


def _edge_pipeline(nch, g_hbm, src_v, dst_v, buf_a, buf_b, sem_a, sem_b, acc):
    pltpu.async_copy(g_hbm.at[src_v.at[0]], buf_a, sem_a)

    def body(p, carry):
        j = 2 * p
        pltpu.async_copy(g_hbm.at[src_v.at[j + 1]], buf_b, sem_b)
        pltpu.make_async_copy(g_hbm.at[src_v.at[j]], buf_a, sem_a).wait()
        pltpu.sync_copy(buf_a, acc.at[dst_v.at[j]], add=True)

        @pl.when(j + 2 < nch)
        def _prefetch():
            pltpu.async_copy(g_hbm.at[src_v.at[j + 2]], buf_a, sem_a)

        pltpu.make_async_copy(g_hbm.at[src_v.at[j + 1]], buf_b, sem_b).wait()
        pltpu.sync_copy(buf_b, acc.at[dst_v.at[j + 1]], add=True)
        return carry

    lax.fori_loop(0, nch // 2, body, 0)


@functools.partial(
    pl.kernel,
    out_type=jax.ShapeDtypeStruct((NC, NPAD, N_CLASSES), jnp.float32),
    mesh=_mesh,
    scratch_types=[
        pltpu.VMEM((NCH0, CHUNK), jnp.int32),
        pltpu.VMEM((NCH0, CHUNK), jnp.int32),
        pltpu.VMEM((CHUNK, N_CLASSES), jnp.float32),
        pltpu.VMEM((CHUNK, N_CLASSES), jnp.float32),
        pltpu.SemaphoreType.DMA,
        pltpu.SemaphoreType.DMA,
        pltpu.VMEM_SHARED((NPAD, N_CLASSES), jnp.float32),
    ],
    compiler_params=pltpu.CompilerParams(use_tc_tiling_on_sc=False),
)
def _hop_kernel(g_hbm, src_hbm, dst_hbm, zeros_hbm, out_hbm,
                src_v, dst_v, buf_a, buf_b, sem_a, sem_b, acc):
    c = lax.axis_index("c")
    s = lax.axis_index("s")
    base = _chunk_base(c, s)
    pltpu.sync_copy(zeros_hbm.at[pl.ds(s * RPT, RPT)], acc.at[pl.ds(s * RPT, RPT)])

    @pl.when(c == 0)
    def _load0():
        pltpu.sync_copy(src_hbm.at[pl.ds(base, NCH0)], src_v)
        pltpu.sync_copy(dst_hbm.at[pl.ds(base, NCH0)], dst_v)

    @pl.when(c == 1)
    def _load1():
        pltpu.sync_copy(src_hbm.at[pl.ds(base, NCH1)], src_v.at[pl.ds(0, NCH1)])
        pltpu.sync_copy(dst_hbm.at[pl.ds(base, NCH1)], dst_v.at[pl.ds(0, NCH1)])

    plsc.subcore_barrier()

    @pl.when(c == 0)
    def _pipe0():
        _edge_pipeline(NCH0, g_hbm, src_v, dst_v, buf_a, buf_b, sem_a, sem_b, acc)

    @pl.when(c == 1)
    def _pipe1():
        _edge_pipeline(NCH1, g_hbm, src_v, dst_v, buf_a, buf_b, sem_a, sem_b, acc)

    plsc.subcore_barrier()
    pltpu.sync_copy(acc.at[pl.ds(s * RPT, RPT)], out_hbm.at[c, pl.ds(s * RPT, RPT)])


_RB = 640


def _scale_body(degp_ref, x_ref, w_ref, g0_ref, dinv_ref):
    deg = degp_ref[0, :, 0:1] + degp_ref[1, :, 0:1] + 1.0
    dinv = lax.rsqrt(deg)
    h0 = jnp.dot(x_ref[...], w_ref[...], preferred_element_type=jnp.float32)
    g0_ref[...] = h0 * dinv
    dinv_ref[...] = jnp.broadcast_to(dinv, (_RB, N_CLASSES))


_scale_call = pl.pallas_call(
    _scale_body,
    grid=(NPAD // _RB,),
    in_specs=[
        pl.BlockSpec((NC, _RB, 16), lambda i: (0, i, 0)),
        pl.BlockSpec((_RB, D_FEAT), lambda i: (i, 0)),
        pl.BlockSpec((D_FEAT, N_CLASSES), lambda i: (0, 0)),
    ],
    out_specs=[
        pl.BlockSpec((_RB, N_CLASSES), lambda i: (i, 0)),
        pl.BlockSpec((_RB, N_CLASSES), lambda i: (i, 0)),
    ],
    out_shape=[
        jax.ShapeDtypeStruct((NPAD, N_CLASSES), jnp.float32),
        jax.ShapeDtypeStruct((NPAD, N_CLASSES), jnp.float32),
    ],
)


def _combine_body(p_ref, g0_ref, dinv_ref, g1_ref):
    t = p_ref[0] + p_ref[1] + g0_ref[...]
    dv = dinv_ref[...]
    g1_ref[...] = dv * dv * t


_combine_call = pl.pallas_call(
    _combine_body,
    grid=(NPAD // _RB,),
    in_specs=[
        pl.BlockSpec((NC, _RB, N_CLASSES), lambda i: (0, i, 0)),
        pl.BlockSpec((_RB, N_CLASSES), lambda i: (i, 0)),
        pl.BlockSpec((_RB, N_CLASSES), lambda i: (i, 0)),
    ],
    out_specs=pl.BlockSpec((_RB, N_CLASSES), lambda i: (i, 0)),
    out_shape=jax.ShapeDtypeStruct((NPAD, N_CLASSES), jnp.float32),
)

_FB = 400


def _final_body(q_ref, g1_ref, dinv_ref, b_ref, out_ref):
    logits = dinv_ref[...] * (q_ref[0] + q_ref[1] + g1_ref[...]) + b_ref[...]
    m = jnp.max(logits, axis=1, keepdims=True)
    ex = jnp.exp(logits - m)
    out_ref[...] = logits - m - jnp.log(jnp.sum(ex, axis=1, keepdims=True))


_final_call = pl.pallas_call(
    _final_body,
    grid=(N_NODES // _FB,),
    in_specs=[
        pl.BlockSpec((NC, _FB, N_CLASSES), lambda i: (0, i, 0)),
        pl.BlockSpec((_FB, N_CLASSES), lambda i: (i, 0)),
        pl.BlockSpec((_FB, N_CLASSES), lambda i: (i, 0)),
        pl.BlockSpec((1, N_CLASSES), lambda i: (0, 0)),
    ],
    out_specs=pl.BlockSpec((_FB, N_CLASSES), lambda i: (i, 0)),
    out_shape=jax.ShapeDtypeStruct((N_NODES, N_CLASSES), jnp.float32),
)


def kernel(x, edge_index, W, b):
    src = edge_index[0].astype(jnp.int32)
    dst = edge_index[1].astype(jnp.int32)
    pad_e = EPAD - N_EDGES
    src_p = jnp.concatenate(
        [src, jnp.zeros((pad_e,), jnp.int32)]).reshape(TCH, CHUNK)
    dst_p = jnp.concatenate(
        [dst, jnp.full((pad_e,), TRASH, jnp.int32)]).reshape(TCH, CHUNK)
    x_p = jnp.concatenate(
        [x, jnp.zeros((NPAD - N_NODES, D_FEAT), jnp.float32)])
    ones16 = jnp.ones((CHUNK, 16), jnp.float32)
    zeros16 = jnp.zeros((NPAD, 16), jnp.float32)
    zeros64 = jnp.zeros((NPAD, N_CLASSES), jnp.float32)

    degp = _deg_kernel(dst_p, ones16, zeros16)
    g0, dinv = _scale_call(degp, x_p, W)
    p = _hop_kernel(g0, src_p, dst_p, zeros64)
    g1 = _combine_call(p, g0, dinv)
    q = _hop_kernel(g1, src_p, dst_p, zeros64)
    return _final_call(q, g1, dinv, b.reshape(1, N_CLASSES))
